# Initial kernel scaffold; baseline (speedup 1.0000x reference)
import jax
import jax.numpy as jnp
from jax.experimental import pallas as pl


def kernel(x_feat, edge_index, edge_attr, bases, W_e, b_e, W_p1, b_p1, W_p2, b_p2, W_f1, b_f1, W_f2, b_f2, gamma, beta):
    raise NotImplementedError("write your pallas kernel here")


# speedup vs baseline: 1.3225x; 1.3225x over previous
import functools

import jax
import jax.numpy as jnp
from jax import lax
from jax.experimental import pallas as pl
from jax.experimental.pallas import tpu as pltpu
from jax.experimental.pallas import tpu_sc as plsc

N = 10000
E = 320000
H = 128
HE = 256

NC = 2
NS = 16
NW = NC * NS
B = 80
EPT = E // NW
GBLK = EPT // B
EPS = E // NS
NBLK = EPS // B
N2 = 10240
RPS = N2 // NS

RB = 1000
NRB = N // RB
EB = 512
NEB = E // EB


def _mesh():
    return plsc.VectorSubcoreMesh(
        core_axis_name="c", subcore_axis_name="s", num_cores=NC, num_subcores=NS)


def _gather_body(x_hbm, src_hbm, xg_hbm, srcv, rows, sem):
    w = lax.axis_index("c") * NS + lax.axis_index("s")
    ebase = w * EPT

    def step(i, carry):
        e0 = ebase + i * B
        pltpu.sync_copy(src_hbm.at[pl.ds(e0, B)], srcv)
        pltpu.async_copy(x_hbm.at[srcv], rows, sem).wait()
        pltpu.sync_copy(rows, xg_hbm.at[pl.ds(e0, B)])
        return carry

    lax.fori_loop(0, GBLK, step, 0)


def _make_gather(x_feat, src):
    fn = functools.partial(
        pl.kernel,
        out_type=jax.ShapeDtypeStruct((E, H), jnp.float32),
        mesh=_mesh(),
        scratch_types=[
            pltpu.VMEM((B,), jnp.int32),
            pltpu.VMEM((B, H), jnp.float32),
            pltpu.SemaphoreType.DMA,
        ],
    )(_gather_body)
    return fn(x_feat, src)


def _v_body(xg_ref, attr_ref, bases_ref, we_ref, be_ref, wp1_ref, bp1_ref,
            out_ref):
    e = attr_ref[...] @ we_ref[...] + be_ref[...]
    pos = xg_ref[...] + e
    out_ref[...] = jnp.maximum(pos @ wp1_ref[...] + bp1_ref[...], 0.0) \
        * bases_ref[...]


def _make_v(xg, edge_attr, bases, W_e, b_e, W_p1, b_p1):
    return pl.pallas_call(
        _v_body,
        grid=(NEB, NC),
        in_specs=[
            pl.BlockSpec((EB, H), lambda i, h: (i, 0)),
            pl.BlockSpec((EB, 7), lambda i, h: (i, 0)),
            pl.BlockSpec((EB, H), lambda i, h: (i, h)),
            pl.BlockSpec((7, H), lambda i, h: (0, 0)),
            pl.BlockSpec((1, H), lambda i, h: (0, 0)),
            pl.BlockSpec((H, H), lambda i, h: (0, h)),
            pl.BlockSpec((1, H), lambda i, h: (0, h)),
        ],
        out_specs=pl.BlockSpec((EB, H), lambda i, h: (h * NEB + i, 0)),
        out_shape=jax.ShapeDtypeStruct((NC * E, H), jnp.float32),
    )(xg, edge_attr, bases, W_e, b_e.reshape(1, H), W_p1, b_p1.reshape(1, HE))


def _scatter_body(v_hbm, dst_hbm, zeros_hbm, out_hbm, dstv, vbuf, aggr_sp):
    c = lax.axis_index("c")
    s = lax.axis_index("s")
    row0 = s * RPS
    pltpu.sync_copy(zeros_hbm, aggr_sp.at[pl.ds(row0, RPS)])
    plsc.subcore_barrier()

    ebase = s * EPS

    def step(i, carry):
        e0 = ebase + i * B
        pltpu.sync_copy(dst_hbm.at[pl.ds(e0, B)], dstv)
        pltpu.sync_copy(v_hbm.at[pl.ds(c * E + e0, B)], vbuf)
        pltpu.sync_copy(vbuf, aggr_sp.at[dstv], add=True)
        return carry

    lax.fori_loop(0, NBLK, step, 0)
    plsc.subcore_barrier()
    pltpu.sync_copy(aggr_sp.at[pl.ds(row0, RPS)], out_hbm.at[c, pl.ds(row0, RPS)])


def _make_aggr(v_st, dst, zeros):
    fn = functools.partial(
        pl.kernel,
        out_type=jax.ShapeDtypeStruct((NC, N2, H), jnp.float32),
        mesh=_mesh(),
        scratch_types=[
            pltpu.VMEM((B,), jnp.int32),
            pltpu.VMEM((B, H), jnp.float32),
            pltpu.VMEM_SHARED((N2, H), jnp.float32),
        ],
    )(_scatter_body)
    return fn(v_st, dst, zeros)


def _ffn_body(alo_ref, ahi_ref, x_ref, wp2_ref, bp2_ref, wf1_ref, bf1_ref,
              wf2_ref, bf2_ref, h2_ref, xtx_ref, sh_ref, ssq_ref):
    i = pl.program_id(0)
    aggr = jnp.concatenate([alo_ref[0], ahi_ref[0]], axis=1)
    y = aggr + jnp.maximum(x_ref[...] @ wp2_ref[...] + bp2_ref[...], 0.0)

    @pl.when(i == 0)
    def _():
        xtx_ref[...] = jnp.zeros_like(xtx_ref)
        sh_ref[...] = jnp.zeros_like(sh_ref)
        ssq_ref[...] = jnp.zeros_like(ssq_ref)

    xtx_ref[...] += lax.dot_general(y, y, (((0,), (0,)), ((), ())))
    h = jnp.maximum(y @ wf1_ref[...] + bf1_ref[...], 0.0)
    h2 = jnp.maximum(h @ wf2_ref[...] + bf2_ref[...], 0.0)
    h2_ref[...] = h2
    sh_ref[...] += jnp.sum(h2, axis=0, keepdims=True)
    ssq_ref[...] += jnp.sum(h2 * h2, axis=0, keepdims=True)


def _make_ffn(aggr2, x_feat, W_p2, b_p2, W_f1, b_f1, W_f2, b_f2):
    return pl.pallas_call(
        _ffn_body,
        grid=(NRB,),
        in_specs=[
            pl.BlockSpec((1, RB, H), lambda i: (0, i, 0)),
            pl.BlockSpec((1, RB, H), lambda i: (1, i, 0)),
            pl.BlockSpec((RB, H), lambda i: (i, 0)),
            pl.BlockSpec((H, HE), lambda i: (0, 0)),
            pl.BlockSpec((1, HE), lambda i: (0, 0)),
            pl.BlockSpec((HE, HE), lambda i: (0, 0)),
            pl.BlockSpec((1, HE), lambda i: (0, 0)),
            pl.BlockSpec((HE, H), lambda i: (0, 0)),
            pl.BlockSpec((1, H), lambda i: (0, 0)),
        ],
        out_specs=[
            pl.BlockSpec((RB, H), lambda i: (i, 0)),
            pl.BlockSpec((HE, HE), lambda i: (0, 0)),
            pl.BlockSpec((1, H), lambda i: (0, 0)),
            pl.BlockSpec((1, H), lambda i: (0, 0)),
        ],
        out_shape=[
            jax.ShapeDtypeStruct((N, H), jnp.float32),
            jax.ShapeDtypeStruct((HE, HE), jnp.float32),
            jax.ShapeDtypeStruct((1, H), jnp.float32),
            jax.ShapeDtypeStruct((1, H), jnp.float32),
        ],
    )(aggr2, aggr2, x_feat, W_p2, b_p2.reshape(1, HE), W_f1,
      b_f1.reshape(1, HE), W_f2, b_f2.reshape(1, H))


def _bn_body(h2_ref, sh_ref, ssq_ref, g_ref, b_ref, out_ref):
    mean = sh_ref[...] / N
    var = ssq_ref[...] / N - mean * mean
    rstd = lax.rsqrt(var + 1e-5)
    out_ref[...] = (h2_ref[...] - mean) * (rstd * g_ref[...]) + b_ref[...]


def _make_bn(h2, sh, ssq, gamma, beta):
    return pl.pallas_call(
        _bn_body,
        grid=(NRB,),
        in_specs=[
            pl.BlockSpec((RB, H), lambda i: (i, 0)),
            pl.BlockSpec((1, H), lambda i: (0, 0)),
            pl.BlockSpec((1, H), lambda i: (0, 0)),
            pl.BlockSpec((1, H), lambda i: (0, 0)),
            pl.BlockSpec((1, H), lambda i: (0, 0)),
        ],
        out_specs=pl.BlockSpec((RB, H), lambda i: (i, 0)),
        out_shape=jax.ShapeDtypeStruct((N, H), jnp.float32),
    )(h2, sh, ssq, gamma.reshape(1, H), beta.reshape(1, H))


def kernel(x_feat, edge_index, edge_attr, bases, W_e, b_e, W_p1, b_p1,
           W_p2, b_p2, W_f1, b_f1, W_f2, b_f2, gamma, beta):
    src = edge_index[0]
    dst = edge_index[1]
    zeros = jnp.zeros((RPS, H), jnp.float32)

    xg = _make_gather(x_feat, src)
    v_st = _make_v(xg, edge_attr, bases, W_e, b_e, W_p1, b_p1)
    aggr2 = _make_aggr(v_st, dst, zeros)
    h2, xtx, sh, ssq = _make_ffn(aggr2, x_feat, W_p2, b_p2, W_f1, b_f1,
                                 W_f2, b_f2)
    out = _make_bn(h2, sh, ssq, gamma, beta)
    return (out, xtx)


# speedup vs baseline: 2.9320x; 2.2170x over previous
import functools

import jax
import jax.numpy as jnp
from jax import lax
from jax.experimental import pallas as pl
from jax.experimental.pallas import tpu as pltpu
from jax.experimental.pallas import tpu_sc as plsc

N = 10000
E = 320000
H = 128
HE = 256

NC = 2
NS = 16
NW = NC * NS
B = 80
UNR = 5
UNS = 2
EPT = E // NW
GBLK = EPT // B
EPS = E // NS
NBLK = EPS // B
N2 = 10240
RPS = N2 // NS

RB = 1000
NRB = N // RB
EB = 2000
NEB = E // EB


def _mesh():
    return plsc.VectorSubcoreMesh(
        core_axis_name="c", subcore_axis_name="s", num_cores=NC, num_subcores=NS)


def _gather_body(x_hbm, src_hbm, xg_hbm, srcv, *bufs):
    rows = bufs[0:UNR]
    gsem = bufs[UNR:2 * UNR]
    wsem = bufs[2 * UNR:3 * UNR]
    w = lax.axis_index("c") * NS + lax.axis_index("s")
    ebase = w * EPT
    pltpu.sync_copy(src_hbm.at[pl.ds(ebase, EPT)], srcv)

    def outer(it, carry):
        base = it * UNR
        gds = []
        for b in range(UNR):
            blk = base + b
            gds.append(pltpu.async_copy(
                x_hbm.at[srcv.at[pl.ds(blk * B, B)]], rows[b], gsem[b]))
        wds = []
        for b in range(UNR):
            blk = base + b
            gds[b].wait()
            wds.append(pltpu.async_copy(
                rows[b], xg_hbm.at[pl.ds(ebase + blk * B, B)], wsem[b]))
        for b in range(UNR):
            wds[b].wait()
        return carry

    lax.fori_loop(0, GBLK // UNR, outer, 0)


def _make_gather(x_feat, src):
    fn = functools.partial(
        pl.kernel,
        out_type=jax.ShapeDtypeStruct((E, H), jnp.float32),
        mesh=_mesh(),
        scratch_types=(
            [pltpu.VMEM((EPT,), jnp.int32)]
            + [pltpu.VMEM((B, H), jnp.float32)] * UNR
            + [pltpu.SemaphoreType.DMA] * (2 * UNR)
        ),
    )(_gather_body)
    return fn(x_feat, src)


def _v_body(xg_ref, attr_ref, bases_ref, we_ref, be_ref, wp1_ref, bp1_ref,
            out_ref):
    e = attr_ref[...] @ we_ref[...] + be_ref[...]
    pos = xg_ref[...] + e
    out_ref[...] = jnp.maximum(pos @ wp1_ref[...] + bp1_ref[...], 0.0) \
        * bases_ref[...]


def _make_v(xg, edge_attr, bases, W_e, b_e, W_p1, b_p1):
    return pl.pallas_call(
        _v_body,
        grid=(NEB,),
        in_specs=[
            pl.BlockSpec((EB, H), lambda i: (i, 0)),
            pl.BlockSpec((EB, 7), lambda i: (i, 0)),
            pl.BlockSpec((EB, HE), lambda i: (i, 0)),
            pl.BlockSpec((7, H), lambda i: (0, 0)),
            pl.BlockSpec((1, H), lambda i: (0, 0)),
            pl.BlockSpec((H, HE), lambda i: (0, 0)),
            pl.BlockSpec((1, HE), lambda i: (0, 0)),
        ],
        out_specs=pl.BlockSpec((EB, HE), lambda i: (i, 0)),
        out_shape=jax.ShapeDtypeStruct((E, HE), jnp.float32),
    )(xg, edge_attr, bases, W_e, b_e.reshape(1, H), W_p1, b_p1.reshape(1, HE))


def _scatter_body(v_hbm, dst_hbm, zeros_hbm, out_hbm, *bufs):
    dbufs = bufs[0:UNS]
    vbufs = bufs[UNS:2 * UNS]
    isem = bufs[2 * UNS:3 * UNS]
    vsem = bufs[3 * UNS:4 * UNS]
    aggr_sp = bufs[4 * UNS]
    c = lax.axis_index("c")
    s = lax.axis_index("s")
    row0 = s * RPS
    pltpu.sync_copy(zeros_hbm, aggr_sp.at[pl.ds(row0, RPS)])
    plsc.subcore_barrier()

    ebase = s * EPS
    col0 = c * H

    def outer(it, carry):
        base = it * UNS
        ids, vds = [], []
        for b in range(UNS):
            e0 = ebase + (base + b) * B
            ids.append(pltpu.async_copy(
                dst_hbm.at[pl.ds(e0, B)], dbufs[b], isem[b]))
            vds.append(pltpu.async_copy(
                v_hbm.at[pl.ds(e0, B), pl.ds(col0, H)], vbufs[b], vsem[b]))
        for b in range(UNS):
            ids[b].wait()
            vds[b].wait()
            pltpu.sync_copy(vbufs[b], aggr_sp.at[dbufs[b]], add=True)
        return carry

    lax.fori_loop(0, NBLK // UNS, outer, 0)
    plsc.subcore_barrier()
    pltpu.sync_copy(aggr_sp.at[pl.ds(row0, RPS)], out_hbm.at[c, pl.ds(row0, RPS)])


def _make_aggr(v, dst, zeros):
    fn = functools.partial(
        pl.kernel,
        out_type=jax.ShapeDtypeStruct((NC, N2, H), jnp.float32),
        mesh=_mesh(),
        scratch_types=(
            [pltpu.VMEM((B,), jnp.int32)] * UNS
            + [pltpu.VMEM((B, H), jnp.float32)] * UNS
            + [pltpu.SemaphoreType.DMA] * (2 * UNS)
            + [pltpu.VMEM_SHARED((N2, H), jnp.float32)]
        ),
    )(_scatter_body)
    return fn(v, dst, zeros)


def _ffn_body(alo_ref, ahi_ref, x_ref, wp2_ref, bp2_ref, wf1_ref, bf1_ref,
              wf2_ref, bf2_ref, h2_ref, xtx_ref, sh_ref, ssq_ref):
    i = pl.program_id(0)
    aggr = jnp.concatenate([alo_ref[0], ahi_ref[0]], axis=1)
    y = aggr + jnp.maximum(x_ref[...] @ wp2_ref[...] + bp2_ref[...], 0.0)

    @pl.when(i == 0)
    def _():
        xtx_ref[...] = jnp.zeros_like(xtx_ref)
        sh_ref[...] = jnp.zeros_like(sh_ref)
        ssq_ref[...] = jnp.zeros_like(ssq_ref)

    xtx_ref[...] += lax.dot_general(y, y, (((0,), (0,)), ((), ())))
    h = jnp.maximum(y @ wf1_ref[...] + bf1_ref[...], 0.0)
    h2 = jnp.maximum(h @ wf2_ref[...] + bf2_ref[...], 0.0)
    h2_ref[...] = h2
    sh_ref[...] += jnp.sum(h2, axis=0, keepdims=True)
    ssq_ref[...] += jnp.sum(h2 * h2, axis=0, keepdims=True)


def _make_ffn(aggr2, x_feat, W_p2, b_p2, W_f1, b_f1, W_f2, b_f2):
    return pl.pallas_call(
        _ffn_body,
        grid=(NRB,),
        in_specs=[
            pl.BlockSpec((1, RB, H), lambda i: (0, i, 0)),
            pl.BlockSpec((1, RB, H), lambda i: (1, i, 0)),
            pl.BlockSpec((RB, H), lambda i: (i, 0)),
            pl.BlockSpec((H, HE), lambda i: (0, 0)),
            pl.BlockSpec((1, HE), lambda i: (0, 0)),
            pl.BlockSpec((HE, HE), lambda i: (0, 0)),
            pl.BlockSpec((1, HE), lambda i: (0, 0)),
            pl.BlockSpec((HE, H), lambda i: (0, 0)),
            pl.BlockSpec((1, H), lambda i: (0, 0)),
        ],
        out_specs=[
            pl.BlockSpec((RB, H), lambda i: (i, 0)),
            pl.BlockSpec((HE, HE), lambda i: (0, 0)),
            pl.BlockSpec((1, H), lambda i: (0, 0)),
            pl.BlockSpec((1, H), lambda i: (0, 0)),
        ],
        out_shape=[
            jax.ShapeDtypeStruct((N, H), jnp.float32),
            jax.ShapeDtypeStruct((HE, HE), jnp.float32),
            jax.ShapeDtypeStruct((1, H), jnp.float32),
            jax.ShapeDtypeStruct((1, H), jnp.float32),
        ],
    )(aggr2, aggr2, x_feat, W_p2, b_p2.reshape(1, HE), W_f1,
      b_f1.reshape(1, HE), W_f2, b_f2.reshape(1, H))


def _bn_body(h2_ref, sh_ref, ssq_ref, g_ref, b_ref, out_ref):
    mean = sh_ref[...] / N
    var = ssq_ref[...] / N - mean * mean
    rstd = lax.rsqrt(var + 1e-5)
    out_ref[...] = (h2_ref[...] - mean) * (rstd * g_ref[...]) + b_ref[...]


def _make_bn(h2, sh, ssq, gamma, beta):
    return pl.pallas_call(
        _bn_body,
        grid=(NRB,),
        in_specs=[
            pl.BlockSpec((RB, H), lambda i: (i, 0)),
            pl.BlockSpec((1, H), lambda i: (0, 0)),
            pl.BlockSpec((1, H), lambda i: (0, 0)),
            pl.BlockSpec((1, H), lambda i: (0, 0)),
            pl.BlockSpec((1, H), lambda i: (0, 0)),
        ],
        out_specs=pl.BlockSpec((RB, H), lambda i: (i, 0)),
        out_shape=jax.ShapeDtypeStruct((N, H), jnp.float32),
    )(h2, sh, ssq, gamma.reshape(1, H), beta.reshape(1, H))


def kernel(x_feat, edge_index, edge_attr, bases, W_e, b_e, W_p1, b_p1,
           W_p2, b_p2, W_f1, b_f1, W_f2, b_f2, gamma, beta):
    src = edge_index[0]
    dst = edge_index[1]
    zeros = jnp.zeros((RPS, H), jnp.float32)

    xg = _make_gather(x_feat, src)
    v = _make_v(xg, edge_attr, bases, W_e, b_e, W_p1, b_p1)
    aggr2 = _make_aggr(v, dst, zeros)
    h2, xtx, sh, ssq = _make_ffn(aggr2, x_feat, W_p2, b_p2, W_f1, b_f1,
                                 W_f2, b_f2)
    out = _make_bn(h2, sh, ssq, gamma, beta)
    return (out, xtx)


# speedup vs baseline: 3.6955x; 1.2604x over previous
import functools

import jax
import jax.numpy as jnp
from jax import lax
from jax.experimental import pallas as pl
from jax.experimental.pallas import tpu as pltpu
from jax.experimental.pallas import tpu_sc as plsc

N = 10000
E = 320000
H = 128
HE = 256

NC = 2
NS = 16
NW = NC * NS
B = 80
UNR = 5
UNS = 2
N2 = 10240
RPS = N2 // NS

E1 = 153600
E2 = E - E1

RB = 1000
NRB = N // RB
EB = 2560


def _mesh():
    return plsc.VectorSubcoreMesh(
        core_axis_name="c", subcore_axis_name="s", num_cores=NC, num_subcores=NS)


def _gather_chunk(e_off, e_cnt):
    ept = e_cnt // NW
    gblk = ept // B
    assert gblk % UNR == 0

    def body(x_hbm, src_hbm, xg_hbm, srcv, *bufs):
        rows = bufs[0:UNR]
        gsem = bufs[UNR:2 * UNR]
        wsem = bufs[2 * UNR:3 * UNR]
        w = lax.axis_index("c") * NS + lax.axis_index("s")
        base_l = w * ept
        pltpu.sync_copy(src_hbm.at[pl.ds(e_off + base_l, ept)], srcv)

        def outer(it, carry):
            base = it * UNR
            gds = []
            for b in range(UNR):
                blk = base + b
                gds.append(pltpu.async_copy(
                    x_hbm.at[srcv.at[pl.ds(blk * B, B)]], rows[b], gsem[b]))
            wds = []
            for b in range(UNR):
                blk = base + b
                gds[b].wait()
                wds.append(pltpu.async_copy(
                    rows[b], xg_hbm.at[pl.ds(base_l + blk * B, B)], wsem[b]))
            for b in range(UNR):
                wds[b].wait()
            return carry

        lax.fori_loop(0, gblk // UNR, outer, 0)

    return functools.partial(
        pl.kernel,
        out_type=jax.ShapeDtypeStruct((e_cnt, H), jnp.float32),
        mesh=_mesh(),
        scratch_types=(
            [pltpu.VMEM((ept,), jnp.int32)]
            + [pltpu.VMEM((B, H), jnp.float32)] * UNR
            + [pltpu.SemaphoreType.DMA] * (2 * UNR)
        ),
    )(body)


def _v_body(xg_ref, attrt_ref, bases_ref, we_ref, be_ref, wp1_ref, bp1_ref,
            out_ref):
    e = lax.dot_general(attrt_ref[...], we_ref[...],
                        (((0,), (0,)), ((), ()))) + be_ref[...]
    pos = xg_ref[...] + e
    out_ref[...] = jnp.maximum(pos @ wp1_ref[...] + bp1_ref[...], 0.0) \
        * bases_ref[...]


def _make_v(xg, attrt, bases, W_e, b_e, W_p1, b_p1, e_off, e_cnt):
    neb = e_cnt // EB
    offb = e_off // EB
    return pl.pallas_call(
        _v_body,
        grid=(neb,),
        in_specs=[
            pl.BlockSpec((EB, H), lambda i: (i, 0)),
            pl.BlockSpec((7, EB), lambda i: (0, offb + i)),
            pl.BlockSpec((EB, HE), lambda i: (offb + i, 0)),
            pl.BlockSpec((7, H), lambda i: (0, 0)),
            pl.BlockSpec((1, H), lambda i: (0, 0)),
            pl.BlockSpec((H, HE), lambda i: (0, 0)),
            pl.BlockSpec((1, HE), lambda i: (0, 0)),
        ],
        out_specs=pl.BlockSpec((EB, HE), lambda i: (i, 0)),
        out_shape=jax.ShapeDtypeStruct((e_cnt, HE), jnp.float32),
    )(xg, attrt, bases, W_e, b_e.reshape(1, H), W_p1, b_p1.reshape(1, HE))


def _scatter_chunk(e_off, e_cnt):
    eps = e_cnt // NS
    nblk = eps // B
    assert nblk % UNS == 0

    def body(v_hbm, dst_hbm, zeros_hbm, out_hbm, *bufs):
        dbufs = bufs[0:UNS]
        vbufs = bufs[UNS:2 * UNS]
        isem = bufs[2 * UNS:3 * UNS]
        vsem = bufs[3 * UNS:4 * UNS]
        asem = bufs[4 * UNS:5 * UNS]
        aggr_sp = bufs[5 * UNS]
        c = lax.axis_index("c")
        s = lax.axis_index("s")
        row0 = s * RPS
        pltpu.sync_copy(zeros_hbm, aggr_sp.at[pl.ds(row0, RPS)])
        plsc.subcore_barrier()

        base_l = s * eps
        col0 = c * H

        def outer(it, carry):
            base = it * UNS
            ids, vds = [], []
            for b in range(UNS):
                e0 = base_l + (base + b) * B
                ids.append(pltpu.async_copy(
                    dst_hbm.at[pl.ds(e_off + e0, B)], dbufs[b], isem[b]))
                vds.append(pltpu.async_copy(
                    v_hbm.at[pl.ds(e0, B), pl.ds(col0, H)], vbufs[b], vsem[b]))
            ads = []
            for b in range(UNS):
                ids[b].wait()
                vds[b].wait()
                ads.append(pltpu.async_copy(
                    vbufs[b], aggr_sp.at[dbufs[b]], asem[b], add=True))
            for b in range(UNS):
                ads[b].wait()
            return carry

        lax.fori_loop(0, nblk // UNS, outer, 0)
        plsc.subcore_barrier()
        pltpu.sync_copy(aggr_sp.at[pl.ds(row0, RPS)],
                        out_hbm.at[c, pl.ds(row0, RPS)])

    return functools.partial(
        pl.kernel,
        out_type=jax.ShapeDtypeStruct((NC, N2, H), jnp.float32),
        mesh=_mesh(),
        scratch_types=(
            [pltpu.VMEM((B,), jnp.int32)] * UNS
            + [pltpu.VMEM((B, H), jnp.float32)] * UNS
            + [pltpu.SemaphoreType.DMA] * (3 * UNS)
            + [pltpu.VMEM_SHARED((N2, H), jnp.float32)]
        ),
    )(body)


def _ffn_body(a1lo_ref, a1hi_ref, a2lo_ref, a2hi_ref, x_ref, wp2_ref,
              bp2_ref, wf1_ref, bf1_ref, wf2_ref, bf2_ref,
              h2_ref, xtx_ref, sh_ref, ssq_ref):
    i = pl.program_id(0)
    aggr = jnp.concatenate([a1lo_ref[0] + a2lo_ref[0],
                            a1hi_ref[0] + a2hi_ref[0]], axis=1)
    y = aggr + jnp.maximum(x_ref[...] @ wp2_ref[...] + bp2_ref[...], 0.0)

    @pl.when(i == 0)
    def _():
        xtx_ref[...] = jnp.zeros_like(xtx_ref)
        sh_ref[...] = jnp.zeros_like(sh_ref)
        ssq_ref[...] = jnp.zeros_like(ssq_ref)

    xtx_ref[...] += lax.dot_general(y, y, (((0,), (0,)), ((), ())))
    h = jnp.maximum(y @ wf1_ref[...] + bf1_ref[...], 0.0)
    h2 = jnp.maximum(h @ wf2_ref[...] + bf2_ref[...], 0.0)
    h2_ref[...] = h2
    sh_ref[...] += jnp.sum(h2, axis=0, keepdims=True)
    ssq_ref[...] += jnp.sum(h2 * h2, axis=0, keepdims=True)


def _make_ffn(ag1, ag2, x_feat, W_p2, b_p2, W_f1, b_f1, W_f2, b_f2):
    aspec = [
        pl.BlockSpec((1, RB, H), lambda i: (0, i, 0)),
        pl.BlockSpec((1, RB, H), lambda i: (1, i, 0)),
    ]
    return pl.pallas_call(
        _ffn_body,
        grid=(NRB,),
        in_specs=aspec + aspec + [
            pl.BlockSpec((RB, H), lambda i: (i, 0)),
            pl.BlockSpec((H, HE), lambda i: (0, 0)),
            pl.BlockSpec((1, HE), lambda i: (0, 0)),
            pl.BlockSpec((HE, HE), lambda i: (0, 0)),
            pl.BlockSpec((1, HE), lambda i: (0, 0)),
            pl.BlockSpec((HE, H), lambda i: (0, 0)),
            pl.BlockSpec((1, H), lambda i: (0, 0)),
        ],
        out_specs=[
            pl.BlockSpec((RB, H), lambda i: (i, 0)),
            pl.BlockSpec((HE, HE), lambda i: (0, 0)),
            pl.BlockSpec((1, H), lambda i: (0, 0)),
            pl.BlockSpec((1, H), lambda i: (0, 0)),
        ],
        out_shape=[
            jax.ShapeDtypeStruct((N, H), jnp.float32),
            jax.ShapeDtypeStruct((HE, HE), jnp.float32),
            jax.ShapeDtypeStruct((1, H), jnp.float32),
            jax.ShapeDtypeStruct((1, H), jnp.float32),
        ],
    )(ag1, ag1, ag2, ag2, x_feat, W_p2, b_p2.reshape(1, HE), W_f1,
      b_f1.reshape(1, HE), W_f2, b_f2.reshape(1, H))


def _bn_body(h2_ref, sh_ref, ssq_ref, g_ref, b_ref, out_ref):
    mean = sh_ref[...] / N
    var = ssq_ref[...] / N - mean * mean
    rstd = lax.rsqrt(var + 1e-5)
    out_ref[...] = (h2_ref[...] - mean) * (rstd * g_ref[...]) + b_ref[...]


def _make_bn(h2, sh, ssq, gamma, beta):
    return pl.pallas_call(
        _bn_body,
        grid=(NRB,),
        in_specs=[
            pl.BlockSpec((RB, H), lambda i: (i, 0)),
            pl.BlockSpec((1, H), lambda i: (0, 0)),
            pl.BlockSpec((1, H), lambda i: (0, 0)),
            pl.BlockSpec((1, H), lambda i: (0, 0)),
            pl.BlockSpec((1, H), lambda i: (0, 0)),
        ],
        out_specs=pl.BlockSpec((RB, H), lambda i: (i, 0)),
        out_shape=jax.ShapeDtypeStruct((N, H), jnp.float32),
    )(h2, sh, ssq, gamma.reshape(1, H), beta.reshape(1, H))


def kernel(x_feat, edge_index, edge_attr, bases, W_e, b_e, W_p1, b_p1,
           W_p2, b_p2, W_f1, b_f1, W_f2, b_f2, gamma, beta):
    src = edge_index[0]
    dst = edge_index[1]
    attrt = edge_attr.T
    zeros = jnp.zeros((RPS, H), jnp.float32)

    xg1 = _gather_chunk(0, E1)(x_feat, src)
    v1 = _make_v(xg1, attrt, bases, W_e, b_e, W_p1, b_p1, 0, E1)
    ag1 = _scatter_chunk(0, E1)(v1, dst, zeros)
    xg2 = _gather_chunk(E1, E2)(x_feat, src)
    v2 = _make_v(xg2, attrt, bases, W_e, b_e, W_p1, b_p1, E1, E2)
    ag2 = _scatter_chunk(E1, E2)(v2, dst, zeros)

    h2, xtx, sh, ssq = _make_ffn(ag1, ag2, x_feat, W_p2, b_p2, W_f1, b_f1,
                                 W_f2, b_f2)
    out = _make_bn(h2, sh, ssq, gamma, beta)
    return (out, xtx)
